# baseline (speedup 1.0000x reference)
import functools

import jax
import jax.numpy as jnp
from jax import lax
from jax.experimental import pallas as pl
from jax.experimental.pallas import tpu as pltpu
from jax.experimental.pallas import tpu_sc as plsc


def kernel(x, emb_table, pos_table):
    B, S = x.shape
    V, D = emb_table.shape
    scale = float(D) ** 0.5

    info = plsc.get_sparse_core_info()
    NC, NS = info.num_cores, info.num_subcores
    NW = NC * NS
    seqs_per_w = B // NW

    C0 = 104
    C1 = S - C0

    pos = pos_table[:S]

    mesh = plsc.VectorSubcoreMesh(core_axis_name="c", subcore_axis_name="s")

    @functools.partial(
        pl.kernel,
        mesh=mesh,
        compiler_params=pltpu.CompilerParams(use_tc_tiling_on_sc=False),
        out_type=jax.ShapeDtypeStruct((B, S, D), jnp.float32),
        scratch_types=[
            pltpu.VMEM((S,), jnp.int32),
            pltpu.VMEM((S, D), jnp.float32),
            pltpu.VMEM((S, D), jnp.float32),
            pltpu.SemaphoreType.DMA,
        ],
    )
    def emb_kernel(x_hbm, tab_hbm, pos_hbm, out_hbm, idx_v, rows_v, pos_v, sem):
        wid = lax.axis_index("s") * NC + lax.axis_index("c")
        pltpu.sync_copy(pos_hbm, pos_v)

        def seq_body(j, carry):
            seq = wid * seqs_per_w + j
            pltpu.sync_copy(x_hbm.at[seq], idx_v)
            cp0 = pltpu.async_copy(
                tab_hbm.at[idx_v.at[pl.ds(0, C0)]], rows_v.at[pl.ds(0, C0)], sem
            )
            cp1 = pltpu.async_copy(
                tab_hbm.at[idx_v.at[pl.ds(C0, C1)]], rows_v.at[pl.ds(C0, C1)], sem
            )
            cp0.wait()
            cp1.wait()

            def row_body(r, rcarry):
                for c in range(D // 16):
                    sl = pl.ds(c * 16, 16)
                    rows_v[r, sl] = rows_v[r, sl] * scale + pos_v[r, sl]
                return rcarry

            lax.fori_loop(0, S, row_body, 0)
            pltpu.sync_copy(rows_v, out_hbm.at[seq])
            return carry

        lax.fori_loop(0, seqs_per_w, seq_body, 0)

    return emb_kernel(x, emb_table, pos)


# speedup vs baseline: 1.0696x; 1.0696x over previous
import functools

import jax
import jax.numpy as jnp
from jax import lax
from jax.experimental import pallas as pl
from jax.experimental.pallas import tpu as pltpu
from jax.experimental.pallas import tpu_sc as plsc


def kernel(x, emb_table, pos_table):
    B, S = x.shape
    V, D = emb_table.shape
    scale = float(D) ** 0.5
    NVEC = D // 16

    info = plsc.get_sparse_core_info()
    NC, NS = info.num_cores, info.num_subcores
    NW = NC * NS
    seqs_per_w = B // NW

    C0 = 104
    C1 = S - C0

    NB = 4
    RU = 4

    pos = pos_table[:S]

    mesh = plsc.VectorSubcoreMesh(core_axis_name="c", subcore_axis_name="s")

    @functools.partial(
        pl.kernel,
        mesh=mesh,
        compiler_params=pltpu.CompilerParams(use_tc_tiling_on_sc=False),
        out_type=jax.ShapeDtypeStruct((B, S, D), jnp.float32),
        scratch_types=[
            pltpu.VMEM((NB, S), jnp.int32),
            pltpu.VMEM((NB, S, D), jnp.float32),
            pltpu.VMEM((S, D), jnp.float32),
            pltpu.SemaphoreType.DMA((NB,)),
            pltpu.SemaphoreType.DMA((NB,)),
        ],
    )
    def emb_kernel(x_hbm, tab_hbm, pos_hbm, out_hbm, idx_v, rows_v, pos_v,
                   gsem, wsem):
        wid = lax.axis_index("s") * NC + lax.axis_index("c")
        base = wid * seqs_per_w
        pltpu.sync_copy(pos_hbm, pos_v)

        def start_fetch(j):
            b = j % NB
            pltpu.sync_copy(x_hbm.at[base + j], idx_v.at[b])
            g0 = pltpu.async_copy(
                tab_hbm.at[idx_v.at[b, pl.ds(0, C0)]],
                rows_v.at[b, pl.ds(0, C0)],
                gsem.at[b],
            )
            g1 = pltpu.async_copy(
                tab_hbm.at[idx_v.at[b, pl.ds(C0, C1)]],
                rows_v.at[b, pl.ds(C0, C1)],
                gsem.at[b],
            )
            return (g0, g1)

        def compute(b):
            def body(i, carry):
                r = i * RU
                for rr in range(RU):
                    for c in range(NVEC):
                        sl = pl.ds(c * 16, 16)
                        rows_v[b, r + rr, sl] = (
                            rows_v[b, r + rr, sl] * scale + pos_v[r + rr, sl]
                        )
                return carry

            lax.fori_loop(0, S // RU, body, 0)

        gh = [None] * NB
        wh = [None] * NB
        gh[0] = start_fetch(0)
        gh[1] = start_fetch(1)
        for j in range(seqs_per_w):
            b = j % NB
            f = j + 2
            if f < seqs_per_w:
                fb = f % NB
                if wh[fb] is not None:
                    wh[fb].wait()
                gh[fb] = start_fetch(f)
            gh[b][0].wait()
            gh[b][1].wait()
            compute(b)
            wh[b] = pltpu.async_copy(rows_v.at[b], out_hbm.at[base + j],
                                     wsem.at[b])
        for b in range(NB):
            if wh[b] is not None:
                wh[b].wait()

    return emb_kernel(x, emb_table, pos)
